# baseline (speedup 1.0000x reference)
import functools

import jax
import jax.numpy as jnp
from jax import lax
from jax.experimental import pallas as pl
from jax.experimental.pallas import tpu as pltpu
from jax.experimental.pallas import tpu_sc as plsc

L_IDX = (0, 1, 1, 1, 2, 2, 2, 2, 2, 3, 3, 3, 3, 3, 3, 3)
LANES = 16
NC, NS = 2, 16
NW = NC * NS
EB = 128
NFC = 8
PASSES = ((0, 4), (4, 5), (9, 4), (13, 3))
PTW = 144


def _zero_ref(ref):
    z = jnp.zeros((LANES,), jnp.float32)
    for r in range(16):
        for c in range(NFC):
            ref[r, pl.ds(c * LANES, LANES)] = z


def _sc_body(node_feats, edge_attrs, tp_w, sender, receiver, ptable, params,
             out0, out1, p_v, sidx_v, r_v, a_v, s_v, w_v, piece_v, acc, zbuf,
             sem):
    cid = lax.axis_index("c")

    @pl.when(cid == 0)
    def _core0():
        _worker(node_feats, edge_attrs, tp_w, sender, receiver, ptable,
                params, out0, 0, p_v, sidx_v, r_v, a_v, s_v, w_v, piece_v,
                acc, zbuf, sem)

    @pl.when(cid == 1)
    def _core1():
        _worker(node_feats, edge_attrs, tp_w, sender, receiver, ptable,
                params, out1, 1, p_v, sidx_v, r_v, a_v, s_v, w_v, piece_v,
                acc, zbuf, sem)


def _worker(node_feats, edge_attrs, tp_w, sender, receiver, ptable, params,
            out, cid_const, p_v, sidx_v, r_v, a_v, s_v, w_v, piece_v, acc,
            zbuf, sem):
    wid = cid_const * NS + lax.axis_index("s")
    pltpu.sync_copy(params.at[wid], p_v)
    p_vec = p_v[:]
    e_start = p_vec[0]
    e_end = p_vec[1]
    r_start = p_vec[2]
    r_end = p_vec[3]

    _zero_ref(acc)
    _zero_ref(zbuf)

    def fill_rows(lo, hi):
        def f(rr, c):
            pltpu.sync_copy(zbuf, out.at[rr])
            return c
        lax.fori_loop(lo, hi, f, 0)

    def make_pass(pi):
        m_lo, nm = PASSES[pi]
        ls = sorted(set(L_IDX[m_lo:m_lo + nm]))
        nacc = nm * NFC

        def store_accs(accs):
            for j in range(nm):
                for fc in range(NFC):
                    acc[m_lo + j, pl.ds(fc * LANES, LANES)] = accs[j * NFC + fc]

        def load_accs():
            return tuple(acc[m_lo + j, pl.ds(fc * LANES, LANES)]
                         for j in range(nm) for fc in range(NFC))

        def edge_body(i, accs):
            a_row = a_v[i, :]
            s = [s_v[i, pl.ds(fc * LANES, LANES)] for fc in range(NFC)]
            q = {l: [s[fc] * w_v[i, l, pl.ds(fc * LANES, LANES)]
                     for fc in range(NFC)] for l in ls}
            new = list(accs)
            for j in range(nm):
                m = m_lo + j
                a_b = jnp.full((LANES,), a_row[m], jnp.float32)
                ql = q[L_IDX[m]]
                for fc in range(NFC):
                    new[j * NFC + fc] = new[j * NFC + fc] + a_b * ql[fc]
            return tuple(new)

        def run(lo, hi, cont):
            z = jnp.zeros((LANES,), jnp.float32)
            accs = tuple(jnp.where(cont, v, z) for v in load_accs())
            accs = lax.fori_loop(lo, hi, edge_body, accs)
            store_accs(accs)

        return run

    pass_fns = [make_pass(pi) for pi in range(len(PASSES))]

    def block_body(b, r_cur):
        eb = b * EB
        pltpu.sync_copy(sender.at[pl.ds(eb, EB)], sidx_v)
        pltpu.sync_copy(receiver.at[pl.ds(eb, EB)], r_v.at[pl.ds(0, EB)])
        pltpu.sync_copy(edge_attrs.at[pl.ds(eb, EB)], a_v)
        pltpu.sync_copy(tp_w.at[pl.ds(eb, EB)], w_v)
        pltpu.sync_copy(ptable.at[pl.ds(b * PTW, PTW)], piece_v.at[pl.ds(0, PTW)])
        pltpu.async_copy(node_feats.at[sidx_v], s_v, sem).wait()
        lo_i = jnp.maximum(e_start - eb, 0)
        hi_i = jnp.minimum(e_end - eb, EB)
        npieces = piece_v[pl.ds(0, LANES)][0]

        def piece_body(k, r_c):
            st = piece_v[pl.ds(1 + k, LANES)][0]
            en = piece_v[pl.ds(2 + k, LANES)][0]
            active = (st >= lo_i) & (st < hi_i)
            r_seg = r_v[pl.ds(st, LANES)][0]

            @pl.when(active)
            def _piece():
                @pl.when(r_seg != r_c)
                def _boundary():
                    pltpu.sync_copy(acc, out.at[r_c])
                    fill_rows(r_c + 1, r_seg)

                cont = r_seg == r_c
                for fn in pass_fns:
                    fn(st, en, cont)

            return jnp.where(active, r_seg, r_c)

        return lax.fori_loop(0, npieces, piece_body, r_cur)

    b_lo = e_start // EB
    b_hi = (e_end + EB - 1) // EB
    r_cur = lax.fori_loop(b_lo, b_hi, block_body, r_start)

    @pl.when(r_end > r_start)
    def _final():
        pltpu.sync_copy(acc, out.at[r_cur])
        fill_rows(r_cur + 1, r_end)


def kernel(node_feats, edge_attrs, tp_weights, sender_list, receiver_list,
           first_occurences):
    n, f = node_feats.shape
    e = edge_attrs.shape[0]

    fo_ext = jnp.concatenate(
        [first_occurences.astype(jnp.int32),
         jnp.array([e], jnp.int32)])
    targets = (jnp.arange(NW, dtype=jnp.int32) * (e // NW)).astype(jnp.int32)
    b = jnp.searchsorted(fo_ext, targets, side="left").astype(jnp.int32)
    b_ext = jnp.concatenate([b, jnp.array([n], jnp.int32)])
    e_starts = fo_ext[b_ext[:-1]]
    e_ends = fo_ext[b_ext[1:]]
    params = jnp.zeros((NW, 16), jnp.int32)
    params = (params.at[:, 0].set(e_starts)
                    .at[:, 1].set(e_ends)
                    .at[:, 2].set(b_ext[:-1])
                    .at[:, 3].set(b_ext[1:]))

    rl = receiver_list.astype(jnp.int32)
    nb = e // EB
    is_start = jnp.concatenate(
        [jnp.ones((1,), bool), rl[1:] != rl[:-1]])
    local = is_start | (jnp.arange(e) % EB == 0)
    l2 = local.reshape(nb, EB)
    slot = jnp.cumsum(l2.astype(jnp.int32), axis=1) - 1
    npieces = l2.sum(axis=1).astype(jnp.int32)
    starts = jnp.full((nb * (EB + 1) + 1,), EB, jnp.int32)
    rows = jnp.arange(e, dtype=jnp.int32) // EB
    flat_idx = rows * (EB + 1) + slot.reshape(-1)
    flat_idx = jnp.where(local.reshape(-1), flat_idx, nb * (EB + 1))
    starts = starts.at[flat_idx].set(jnp.arange(e, dtype=jnp.int32) % EB)
    starts = starts[:-1].reshape(nb, EB + 1)
    ptable = jnp.concatenate(
        [npieces[:, None], starts,
         jnp.full((nb, PTW - EB - 2), EB, jnp.int32)], axis=1).reshape(-1)

    mesh = plsc.VectorSubcoreMesh(core_axis_name="c", subcore_axis_name="s",
                                  num_cores=NC, num_subcores=NS)
    run = functools.partial(
        pl.kernel,
        out_type=(jax.ShapeDtypeStruct((n, 16, f), jnp.float32),
                  jax.ShapeDtypeStruct((n, 16, f), jnp.float32)),
        mesh=mesh,
        scratch_types=[
            pltpu.VMEM((LANES,), jnp.int32),
            pltpu.VMEM((EB,), jnp.int32),
            pltpu.VMEM((EB + LANES,), jnp.int32),
            pltpu.VMEM((EB, 16), jnp.float32),
            pltpu.VMEM((EB, f), jnp.float32),
            pltpu.VMEM((EB, 4, f), jnp.float32),
            pltpu.VMEM((PTW + LANES,), jnp.int32),
            pltpu.VMEM((16, f), jnp.float32),
            pltpu.VMEM((16, f), jnp.float32),
            pltpu.SemaphoreType.DMA,
        ],
    )(_sc_body)
    out0, out1 = run(node_feats, edge_attrs, tp_weights,
                     sender_list.astype(jnp.int32), rl, ptable, params)
    mid = b_ext[NS]
    owner0 = jnp.arange(n, dtype=jnp.int32) < mid
    return jnp.where(owner0[:, None, None], out0, out1)


# speedup vs baseline: 1.0691x; 1.0691x over previous
import functools

import jax
import jax.numpy as jnp
from jax import lax
from jax.experimental import pallas as pl
from jax.experimental.pallas import tpu as pltpu
from jax.experimental.pallas import tpu_sc as plsc

L_IDX = (0, 1, 1, 1, 2, 2, 2, 2, 2, 3, 3, 3, 3, 3, 3, 3)
LANES = 16
NC, NS = 2, 16
NW = NC * NS
EB = 128
NFC = 8
PASSES = ((0, 4), (4, 5), (9, 4), (13, 3))
PTW = 144


def _zero_ref(ref):
    z = jnp.zeros((LANES,), jnp.float32)
    for r in range(16):
        for c in range(NFC):
            ref[r, pl.ds(c * LANES, LANES)] = z


def _sc_body(node_feats, edge_attrs, tp_w, sender, receiver, ptable, params,
             out, p_v, sidx_v, r_v, a_v, s_v, w_v, piece_v, acc, zbuf, sem):
    wid = lax.axis_index("c") * NS + lax.axis_index("s")
    pltpu.sync_copy(params.at[wid], p_v)
    p_vec = p_v[:]
    e_start = p_vec[0]
    e_end = p_vec[1]
    r_start = p_vec[2]
    r_end = p_vec[3]

    _zero_ref(acc)
    _zero_ref(zbuf)

    def fill_rows(lo, hi):
        def f(rr, c):
            pltpu.sync_copy(zbuf, out.at[rr])
            return c
        lax.fori_loop(lo, hi, f, 0)

    def make_pass(pi):
        m_lo, nm = PASSES[pi]
        ls = sorted(set(L_IDX[m_lo:m_lo + nm]))
        nacc = nm * NFC

        def store_accs(accs):
            for j in range(nm):
                for fc in range(NFC):
                    acc[m_lo + j, pl.ds(fc * LANES, LANES)] = accs[j * NFC + fc]

        def load_accs():
            return tuple(acc[m_lo + j, pl.ds(fc * LANES, LANES)]
                         for j in range(nm) for fc in range(NFC))

        def edge_body(i, accs):
            a_row = a_v[pl.ds(i * 16, LANES)]
            wb = i * 512
            s = [s_v[i, pl.ds(fc * LANES, LANES)] for fc in range(NFC)]
            q = {l: [s[fc] * w_v[pl.ds(wb + l * 128 + fc * LANES, LANES)]
                     for fc in range(NFC)] for l in ls}
            new = list(accs)
            for j in range(nm):
                m = m_lo + j
                a_b = jnp.full((LANES,), a_row[m], jnp.float32)
                ql = q[L_IDX[m]]
                for fc in range(NFC):
                    new[j * NFC + fc] = new[j * NFC + fc] + a_b * ql[fc]
            return tuple(new)

        def run(lo, hi, cont):
            z = jnp.zeros((LANES,), jnp.float32)
            accs = tuple(jnp.where(cont, v, z) for v in load_accs())
            accs = lax.fori_loop(lo, hi, edge_body, accs)
            store_accs(accs)

        return run

    pass_fns = [make_pass(pi) for pi in range(len(PASSES))]

    def block_body(b, r_cur):
        eb = b * EB
        pltpu.sync_copy(sender.at[pl.ds(eb, EB)], sidx_v)
        pltpu.sync_copy(receiver.at[pl.ds(eb, EB)], r_v.at[pl.ds(0, EB)])
        pltpu.sync_copy(edge_attrs.at[pl.ds(eb * 16, EB * 16)], a_v)
        pltpu.sync_copy(tp_w.at[pl.ds(eb * 512, EB * 512)], w_v)
        pltpu.sync_copy(ptable.at[pl.ds(b * PTW, PTW)], piece_v.at[pl.ds(0, PTW)])
        pltpu.async_copy(node_feats.at[sidx_v], s_v, sem).wait()
        lo_i = jnp.maximum(e_start - eb, 0)
        hi_i = jnp.minimum(e_end - eb, EB)
        npieces = piece_v[pl.ds(0, LANES)][0]

        def piece_body(k, r_c):
            st = piece_v[pl.ds(1 + k, LANES)][0]
            en = piece_v[pl.ds(2 + k, LANES)][0]
            active = (st >= lo_i) & (st < hi_i)
            r_seg = r_v[pl.ds(st, LANES)][0]

            @pl.when(active)
            def _piece():
                @pl.when(r_seg != r_c)
                def _boundary():
                    pltpu.sync_copy(acc, out.at[r_c])
                    fill_rows(r_c + 1, r_seg)

                cont = r_seg == r_c
                for fn in pass_fns:
                    fn(st, en, cont)

            return jnp.where(active, r_seg, r_c)

        return lax.fori_loop(0, npieces, piece_body, r_cur)

    b_lo = e_start // EB
    b_hi = (e_end + EB - 1) // EB
    r_cur = lax.fori_loop(b_lo, b_hi, block_body, r_start)

    @pl.when(r_end > r_start)
    def _final():
        pltpu.sync_copy(acc, out.at[r_cur])
        fill_rows(r_cur + 1, r_end)


def kernel(node_feats, edge_attrs, tp_weights, sender_list, receiver_list,
           first_occurences):
    n, f = node_feats.shape
    e = edge_attrs.shape[0]

    fo_ext = jnp.concatenate(
        [first_occurences.astype(jnp.int32),
         jnp.array([e], jnp.int32)])
    targets = (jnp.arange(NW, dtype=jnp.int32) * (e // NW)).astype(jnp.int32)
    b = jnp.searchsorted(fo_ext, targets, side="left").astype(jnp.int32)
    b_ext = jnp.concatenate([b, jnp.array([n], jnp.int32)])
    e_starts = fo_ext[b_ext[:-1]]
    e_ends = fo_ext[b_ext[1:]]
    params = jnp.zeros((NW, 16), jnp.int32)
    params = (params.at[:, 0].set(e_starts)
                    .at[:, 1].set(e_ends)
                    .at[:, 2].set(b_ext[:-1])
                    .at[:, 3].set(b_ext[1:]))

    rl = receiver_list.astype(jnp.int32)
    nb = e // EB
    is_start = jnp.concatenate(
        [jnp.ones((1,), bool), rl[1:] != rl[:-1]])
    local = is_start | (jnp.arange(e) % EB == 0)
    l2 = local.reshape(nb, EB)
    slot = jnp.cumsum(l2.astype(jnp.int32), axis=1) - 1
    npieces = l2.sum(axis=1).astype(jnp.int32)
    starts = jnp.full((nb * (EB + 1) + 1,), EB, jnp.int32)
    rows = jnp.arange(e, dtype=jnp.int32) // EB
    flat_idx = rows * (EB + 1) + slot.reshape(-1)
    flat_idx = jnp.where(local.reshape(-1), flat_idx, nb * (EB + 1))
    starts = starts.at[flat_idx].set(jnp.arange(e, dtype=jnp.int32) % EB)
    starts = starts[:-1].reshape(nb, EB + 1)
    ptable = jnp.concatenate(
        [npieces[:, None], starts,
         jnp.full((nb, PTW - EB - 2), EB, jnp.int32)], axis=1).reshape(-1)

    mesh = plsc.VectorSubcoreMesh(core_axis_name="c", subcore_axis_name="s",
                                  num_cores=NC, num_subcores=NS)
    run = functools.partial(
        pl.kernel,
        out_type=jax.ShapeDtypeStruct((n, 16, f), jnp.float32),
        mesh=mesh,
        scratch_types=[
            pltpu.VMEM((LANES,), jnp.int32),
            pltpu.VMEM((EB,), jnp.int32),
            pltpu.VMEM((EB + LANES,), jnp.int32),
            pltpu.VMEM((EB * 16,), jnp.float32),
            pltpu.VMEM((EB, f), jnp.float32),
            pltpu.VMEM((EB * 4 * f,), jnp.float32),
            pltpu.VMEM((PTW + LANES,), jnp.int32),
            pltpu.VMEM((16, f), jnp.float32),
            pltpu.VMEM((16, f), jnp.float32),
            pltpu.SemaphoreType.DMA,
        ],
    )(_sc_body)
    return run(node_feats, edge_attrs.reshape(-1), tp_weights.reshape(-1),
               sender_list.astype(jnp.int32), rl, ptable, params)


# speedup vs baseline: 1.0691x; 1.0000x over previous
import functools

import jax
import jax.numpy as jnp
from jax import lax
from jax.experimental import pallas as pl
from jax.experimental.pallas import tpu as pltpu
from jax.experimental.pallas import tpu_sc as plsc

L_IDX = (0, 1, 1, 1, 2, 2, 2, 2, 2, 3, 3, 3, 3, 3, 3, 3)
LANES = 16
NC, NS = 2, 16
NW = NC * NS
EB = 128
NFC = 8
PASSES = ((0, 4), (4, 5), (9, 4), (13, 3))
PTW = 144


def _zero_ref(ref):
    z = jnp.zeros((LANES,), jnp.float32)
    for r in range(16):
        for c in range(NFC):
            ref[r, pl.ds(c * LANES, LANES)] = z


def _sc_body(node_feats, edge_attrs, tp_w, sender, receiver, ptable, params,
             out, p_v, sidx_v, r_v, a_v, s_v, w_v, piece_v, acc, zbuf, sem):
    wid = lax.axis_index("c") * NS + lax.axis_index("s")
    pltpu.sync_copy(params.at[pl.ds(wid * 16, 16)], p_v)
    p_vec = p_v[:]
    e_start = p_vec[0]
    e_end = p_vec[1]
    r_start = p_vec[2]
    r_end = p_vec[3]

    _zero_ref(acc)
    _zero_ref(zbuf)

    def fill_rows(lo, hi):
        def f(rr, c):
            pltpu.sync_copy(zbuf, out.at[rr])
            return c
        lax.fori_loop(lo, hi, f, 0)

    def make_pass(pi):
        m_lo, nm = PASSES[pi]
        ls = sorted(set(L_IDX[m_lo:m_lo + nm]))
        nacc = nm * NFC

        def store_accs(accs):
            for j in range(nm):
                for fc in range(NFC):
                    acc[m_lo + j, pl.ds(fc * LANES, LANES)] = accs[j * NFC + fc]

        def load_accs():
            return tuple(acc[m_lo + j, pl.ds(fc * LANES, LANES)]
                         for j in range(nm) for fc in range(NFC))

        def edge_body(i, accs):
            a_row = a_v[pl.ds(i * 16, LANES)]
            wb = i * 512
            s = [s_v[i, pl.ds(fc * LANES, LANES)] for fc in range(NFC)]
            q = {l: [s[fc] * w_v[pl.ds(wb + l * 128 + fc * LANES, LANES)]
                     for fc in range(NFC)] for l in ls}
            new = list(accs)
            for j in range(nm):
                m = m_lo + j
                a_b = jnp.full((LANES,), a_row[m], jnp.float32)
                ql = q[L_IDX[m]]
                for fc in range(NFC):
                    new[j * NFC + fc] = new[j * NFC + fc] + a_b * ql[fc]
            return tuple(new)

        def run(lo, hi, cont):
            z = jnp.zeros((LANES,), jnp.float32)
            accs = tuple(jnp.where(cont, v, z) for v in load_accs())
            accs = lax.fori_loop(lo, hi, edge_body, accs)
            store_accs(accs)

        return run

    pass_fns = [make_pass(pi) for pi in range(len(PASSES))]

    def block_body(b, r_cur):
        eb = b * EB
        pltpu.sync_copy(sender.at[pl.ds(eb, EB)], sidx_v)
        pltpu.sync_copy(receiver.at[pl.ds(eb, EB)], r_v.at[pl.ds(0, EB)])
        pltpu.sync_copy(edge_attrs.at[pl.ds(eb * 16, EB * 16)], a_v)
        pltpu.sync_copy(tp_w.at[pl.ds(eb * 512, EB * 512)], w_v)
        pltpu.sync_copy(ptable.at[pl.ds(b * PTW, PTW)], piece_v.at[pl.ds(0, PTW)])
        pltpu.async_copy(node_feats.at[sidx_v], s_v, sem).wait()
        lo_i = jnp.maximum(e_start - eb, 0)
        hi_i = jnp.minimum(e_end - eb, EB)
        npieces = piece_v[pl.ds(0, LANES)][0]

        def piece_body(k, r_c):
            st = piece_v[pl.ds(1 + k, LANES)][0]
            en = piece_v[pl.ds(2 + k, LANES)][0]
            active = (st >= lo_i) & (st < hi_i)
            r_seg = r_v[pl.ds(st, LANES)][0]

            @pl.when(active)
            def _piece():
                @pl.when(r_seg != r_c)
                def _boundary():
                    pltpu.sync_copy(acc, out.at[r_c])
                    fill_rows(r_c + 1, r_seg)

                cont = r_seg == r_c
                for fn in pass_fns:
                    fn(st, en, cont)

            return jnp.where(active, r_seg, r_c)

        return lax.fori_loop(0, npieces, piece_body, r_cur)

    b_lo = e_start // EB
    b_hi = (e_end + EB - 1) // EB
    r_cur = lax.fori_loop(b_lo, b_hi, block_body, r_start)

    @pl.when(r_end > r_start)
    def _final():
        pltpu.sync_copy(acc, out.at[r_cur])
        fill_rows(r_cur + 1, r_end)


def kernel(node_feats, edge_attrs, tp_weights, sender_list, receiver_list,
           first_occurences):
    n, f = node_feats.shape
    e = edge_attrs.shape[0]

    fo_ext = jnp.concatenate(
        [first_occurences.astype(jnp.int32),
         jnp.array([e], jnp.int32)])
    targets = (jnp.arange(NW, dtype=jnp.int32) * (e // NW)).astype(jnp.int32)
    b = jnp.searchsorted(fo_ext, targets, side="left").astype(jnp.int32)
    b_ext = jnp.concatenate([b, jnp.array([n], jnp.int32)])
    e_starts = fo_ext[b_ext[:-1]]
    e_ends = fo_ext[b_ext[1:]]
    params = jnp.zeros((NW, 16), jnp.int32)
    params = (params.at[:, 0].set(e_starts)
                    .at[:, 1].set(e_ends)
                    .at[:, 2].set(b_ext[:-1])
                    .at[:, 3].set(b_ext[1:]))

    rl = receiver_list.astype(jnp.int32)
    nb = e // EB
    is_start = jnp.concatenate(
        [jnp.ones((1,), bool), rl[1:] != rl[:-1]])
    local = is_start | (jnp.arange(e) % EB == 0)
    l2 = local.reshape(nb, EB)
    slot = jnp.cumsum(l2.astype(jnp.int32), axis=1) - 1
    npieces = l2.sum(axis=1).astype(jnp.int32)
    starts = jnp.full((nb * (EB + 1) + 1,), EB, jnp.int32)
    rows = jnp.arange(e, dtype=jnp.int32) // EB
    flat_idx = rows * (EB + 1) + slot.reshape(-1)
    flat_idx = jnp.where(local.reshape(-1), flat_idx, nb * (EB + 1))
    starts = starts.at[flat_idx].set(jnp.arange(e, dtype=jnp.int32) % EB)
    starts = starts[:-1].reshape(nb, EB + 1)
    ptable = jnp.concatenate(
        [npieces[:, None], starts,
         jnp.full((nb, PTW - EB - 2), EB, jnp.int32)], axis=1).reshape(-1)

    mesh = plsc.VectorSubcoreMesh(core_axis_name="c", subcore_axis_name="s",
                                  num_cores=NC, num_subcores=NS)
    run = functools.partial(
        pl.kernel,
        out_type=jax.ShapeDtypeStruct((n, 16, f), jnp.float32),
        mesh=mesh,
        scratch_types=[
            pltpu.VMEM((LANES,), jnp.int32),
            pltpu.VMEM((EB,), jnp.int32),
            pltpu.VMEM((EB + LANES,), jnp.int32),
            pltpu.VMEM((EB * 16,), jnp.float32),
            pltpu.VMEM((EB, f), jnp.float32),
            pltpu.VMEM((EB * 4 * f,), jnp.float32),
            pltpu.VMEM((PTW + LANES,), jnp.int32),
            pltpu.VMEM((16, f), jnp.float32),
            pltpu.VMEM((16, f), jnp.float32),
            pltpu.SemaphoreType.DMA,
        ],
    )(_sc_body)
    ea, tw = lax.optimization_barrier(
        (edge_attrs.reshape(-1), tp_weights.reshape(-1)))
    return run(node_feats, ea, tw, sender_list.astype(jnp.int32), rl, ptable,
               params.reshape(-1))


# speedup vs baseline: 1.1383x; 1.0647x over previous
import functools

import jax
import jax.numpy as jnp
from jax import lax
from jax.experimental import pallas as pl
from jax.experimental.pallas import tpu as pltpu
from jax.experimental.pallas import tpu_sc as plsc

L_IDX = (0, 1, 1, 1, 2, 2, 2, 2, 2, 3, 3, 3, 3, 3, 3, 3)
LANES = 16
NC, NS = 2, 16
NW = NC * NS
EB = 128
NFC = 8
PASSES = ((0, 4), (4, 5), (9, 4), (13, 3))
PTW = 144


def _zero_ref(ref):
    z = jnp.zeros((LANES,), jnp.float32)
    for r in range(16):
        for c in range(NFC):
            ref[r, pl.ds(c * LANES, LANES)] = z


def _sc_body(node_feats, edge_attrs, tp_w, sender, receiver, ptable, params,
             out, p_v, sidx_v, r_v, a_v, s_v, w_v, piece_v, acc, zbuf, sem,
             sem2):
    wid = lax.axis_index("c") * NS + lax.axis_index("s")
    pltpu.sync_copy(params.at[pl.ds(wid * 16, 16)], p_v)
    p_vec = p_v[:]
    e_start = p_vec[0]
    e_end = p_vec[1]
    r_start = p_vec[2]
    r_end = p_vec[3]

    _zero_ref(acc)
    _zero_ref(zbuf)

    def fill_rows(lo, hi):
        def f(rr, c):
            pltpu.sync_copy(zbuf, out.at[rr])
            return c
        lax.fori_loop(lo, hi, f, 0)

    def make_pass(pi):
        m_lo, nm = PASSES[pi]
        ls = sorted(set(L_IDX[m_lo:m_lo + nm]))
        nacc = nm * NFC

        def store_accs(accs):
            for j in range(nm):
                for fc in range(NFC):
                    acc[m_lo + j, pl.ds(fc * LANES, LANES)] = accs[j * NFC + fc]

        def load_accs():
            return tuple(acc[m_lo + j, pl.ds(fc * LANES, LANES)]
                         for j in range(nm) for fc in range(NFC))

        def edge_body(i, accs):
            a_row = a_v[pl.ds(i * 16, LANES)]
            wb = i * 512
            s = [s_v[i, pl.ds(fc * LANES, LANES)] for fc in range(NFC)]
            q = {l: [s[fc] * w_v[pl.ds(wb + l * 128 + fc * LANES, LANES)]
                     for fc in range(NFC)] for l in ls}
            new = list(accs)
            for j in range(nm):
                m = m_lo + j
                a_b = jnp.full((LANES,), a_row[m], jnp.float32)
                ql = q[L_IDX[m]]
                for fc in range(NFC):
                    new[j * NFC + fc] = new[j * NFC + fc] + a_b * ql[fc]
            return tuple(new)

        def run(lo, hi, cont):
            z = jnp.zeros((LANES,), jnp.float32)
            accs = tuple(jnp.where(cont, v, z) for v in load_accs())
            accs = lax.fori_loop(lo, hi, edge_body, accs)
            store_accs(accs)

        return run

    pass_fns = [make_pass(pi) for pi in range(len(PASSES))]

    def block_body(b, r_cur):
        eb = b * EB
        c_idx = pltpu.async_copy(sender.at[pl.ds(eb, EB)], sidx_v, sem2)
        c_r = pltpu.async_copy(receiver.at[pl.ds(eb, EB)],
                               r_v.at[pl.ds(0, EB)], sem)
        c_a = pltpu.async_copy(edge_attrs.at[pl.ds(eb * 16, EB * 16)], a_v,
                               sem)
        c_w = pltpu.async_copy(tp_w.at[pl.ds(eb * 512, EB * 512)], w_v, sem)
        c_p = pltpu.async_copy(ptable.at[pl.ds(b * PTW, PTW)],
                               piece_v.at[pl.ds(0, PTW)], sem)
        c_idx.wait()
        c_s = pltpu.async_copy(node_feats.at[sidx_v], s_v, sem2)
        c_r.wait()
        c_a.wait()
        c_w.wait()
        c_p.wait()
        c_s.wait()
        lo_i = jnp.maximum(e_start - eb, 0)
        hi_i = jnp.minimum(e_end - eb, EB)
        npieces = piece_v[pl.ds(0, LANES)][0]

        def piece_body(k, r_c):
            st = piece_v[pl.ds(1 + k, LANES)][0]
            en = piece_v[pl.ds(2 + k, LANES)][0]
            active = (st >= lo_i) & (st < hi_i)
            r_seg = r_v[pl.ds(st, LANES)][0]

            @pl.when(active)
            def _piece():
                @pl.when(r_seg != r_c)
                def _boundary():
                    pltpu.sync_copy(acc, out.at[r_c])
                    fill_rows(r_c + 1, r_seg)

                cont = r_seg == r_c
                for fn in pass_fns:
                    fn(st, en, cont)

            return jnp.where(active, r_seg, r_c)

        return lax.fori_loop(0, npieces, piece_body, r_cur)

    b_lo = e_start // EB
    b_hi = (e_end + EB - 1) // EB
    r_cur = lax.fori_loop(b_lo, b_hi, block_body, r_start)

    @pl.when(r_end > r_start)
    def _final():
        pltpu.sync_copy(acc, out.at[r_cur])
        fill_rows(r_cur + 1, r_end)


def kernel(node_feats, edge_attrs, tp_weights, sender_list, receiver_list,
           first_occurences):
    n, f = node_feats.shape
    e = edge_attrs.shape[0]

    fo_ext = jnp.concatenate(
        [first_occurences.astype(jnp.int32),
         jnp.array([e], jnp.int32)])
    targets = (jnp.arange(NW, dtype=jnp.int32) * (e // NW)).astype(jnp.int32)
    b = jnp.searchsorted(fo_ext, targets, side="left").astype(jnp.int32)
    b_ext = jnp.concatenate([b, jnp.array([n], jnp.int32)])
    e_starts = fo_ext[b_ext[:-1]]
    e_ends = fo_ext[b_ext[1:]]
    params = jnp.zeros((NW, 16), jnp.int32)
    params = (params.at[:, 0].set(e_starts)
                    .at[:, 1].set(e_ends)
                    .at[:, 2].set(b_ext[:-1])
                    .at[:, 3].set(b_ext[1:]))

    rl = receiver_list.astype(jnp.int32)
    nb = e // EB
    is_start = jnp.concatenate(
        [jnp.ones((1,), bool), rl[1:] != rl[:-1]])
    local = is_start | (jnp.arange(e) % EB == 0)
    l2 = local.reshape(nb, EB)
    slot = jnp.cumsum(l2.astype(jnp.int32), axis=1) - 1
    npieces = l2.sum(axis=1).astype(jnp.int32)
    starts = jnp.full((nb * (EB + 1) + 1,), EB, jnp.int32)
    rows = jnp.arange(e, dtype=jnp.int32) // EB
    flat_idx = rows * (EB + 1) + slot.reshape(-1)
    flat_idx = jnp.where(local.reshape(-1), flat_idx, nb * (EB + 1))
    starts = starts.at[flat_idx].set(jnp.arange(e, dtype=jnp.int32) % EB)
    starts = starts[:-1].reshape(nb, EB + 1)
    ptable = jnp.concatenate(
        [npieces[:, None], starts,
         jnp.full((nb, PTW - EB - 2), EB, jnp.int32)], axis=1).reshape(-1)

    mesh = plsc.VectorSubcoreMesh(core_axis_name="c", subcore_axis_name="s",
                                  num_cores=NC, num_subcores=NS)
    run = functools.partial(
        pl.kernel,
        out_type=jax.ShapeDtypeStruct((n, 16, f), jnp.float32),
        mesh=mesh,
        scratch_types=[
            pltpu.VMEM((LANES,), jnp.int32),
            pltpu.VMEM((EB,), jnp.int32),
            pltpu.VMEM((EB + LANES,), jnp.int32),
            pltpu.VMEM((EB * 16,), jnp.float32),
            pltpu.VMEM((EB, f), jnp.float32),
            pltpu.VMEM((EB * 4 * f,), jnp.float32),
            pltpu.VMEM((PTW + LANES,), jnp.int32),
            pltpu.VMEM((16, f), jnp.float32),
            pltpu.VMEM((16, f), jnp.float32),
            pltpu.SemaphoreType.DMA,
            pltpu.SemaphoreType.DMA,
        ],
    )(_sc_body)
    ea, tw = lax.optimization_barrier(
        (edge_attrs.reshape(-1), tp_weights.reshape(-1)))
    return run(node_feats, ea, tw, sender_list.astype(jnp.int32), rl, ptable,
               params.reshape(-1))


# speedup vs baseline: 1.1749x; 1.0322x over previous
import functools

import jax
import jax.numpy as jnp
from jax import lax
from jax.experimental import pallas as pl
from jax.experimental.pallas import tpu as pltpu
from jax.experimental.pallas import tpu_sc as plsc

L_IDX = (0, 1, 1, 1, 2, 2, 2, 2, 2, 3, 3, 3, 3, 3, 3, 3)
LANES = 16
NC, NS = 2, 16
NW = NC * NS
EB = 64
NFC = 8
PASSES = ((0, 4), (4, 5), (9, 4), (13, 3))
PTW = 80


def _zero_ref(ref):
    z = jnp.zeros((LANES,), jnp.float32)
    for r in range(16):
        for c in range(NFC):
            ref[r, pl.ds(c * LANES, LANES)] = z


def _sc_body(node_feats, edge_attrs, tp_w, sender, receiver, ptable, params,
             out, p_v, acc, zbuf,
             sidx0, r0, a0, s0, w0, pv0, sa0, sb0, sc0,
             sidx1, r1, a1, s1, w1, pv1, sa1, sb1, sc1):
    bufs = ((sidx0, r0, a0, s0, w0, pv0, sa0, sb0, sc0),
            (sidx1, r1, a1, s1, w1, pv1, sa1, sb1, sc1))

    wid = lax.axis_index("c") * NS + lax.axis_index("s")
    pltpu.sync_copy(params.at[pl.ds(wid * 16, 16)], p_v)
    p_vec = p_v[:]
    e_start = p_vec[0]
    e_end = p_vec[1]
    r_start = p_vec[2]
    r_end = p_vec[3]

    _zero_ref(acc)
    _zero_ref(zbuf)

    def fill_rows(lo, hi):
        def f(rr, c):
            pltpu.sync_copy(zbuf, out.at[rr])
            return c
        lax.fori_loop(lo, hi, f, 0)

    def lin_descs(b, buf):
        sidx_v, r_v, a_v, s_v, w_v, piece_v, sa, sb, sc = buf
        eb = b * EB
        return (
            pltpu.make_async_copy(sender.at[pl.ds(eb, EB)], sidx_v, sa),
            pltpu.make_async_copy(receiver.at[pl.ds(eb, EB)],
                                  r_v.at[pl.ds(0, EB)], sb),
            pltpu.make_async_copy(edge_attrs.at[pl.ds(eb * 16, EB * 16)],
                                  a_v, sb),
            pltpu.make_async_copy(tp_w.at[pl.ds(eb * 512, EB * 512)], w_v,
                                  sb),
            pltpu.make_async_copy(ptable.at[pl.ds(b * PTW, PTW)],
                                  piece_v.at[pl.ds(0, PTW)], sb),
        )

    def gather_desc(buf):
        sidx_v, r_v, a_v, s_v, w_v, piece_v, sa, sb, sc = buf
        return pltpu.make_async_copy(node_feats.at[sidx_v], s_v, sc)

    def prefetch_linear(b, buf):
        for d in lin_descs(b, buf):
            d.start()

    def wait_sidx_start_gather(b, buf):
        lin_descs(b, buf)[0].wait()
        gather_desc(buf).start()

    def wait_rest(b, buf):
        for d in lin_descs(b, buf)[1:]:
            d.wait()
        gather_desc(buf).wait()

    def make_pass(pi, a_v, s_v, w_v):
        m_lo, nm = PASSES[pi]
        ls = sorted(set(L_IDX[m_lo:m_lo + nm]))
        nacc = nm * NFC

        def store_accs(accs):
            for j in range(nm):
                for fc in range(NFC):
                    acc[m_lo + j, pl.ds(fc * LANES, LANES)] = accs[j * NFC + fc]

        def load_accs():
            return tuple(acc[m_lo + j, pl.ds(fc * LANES, LANES)]
                         for j in range(nm) for fc in range(NFC))

        def edge_body(i, accs):
            a_row = a_v[pl.ds(i * 16, LANES)]
            wb = i * 512
            s = [s_v[i, pl.ds(fc * LANES, LANES)] for fc in range(NFC)]
            q = {l: [s[fc] * w_v[pl.ds(wb + l * 128 + fc * LANES, LANES)]
                     for fc in range(NFC)] for l in ls}
            new = list(accs)
            for j in range(nm):
                m = m_lo + j
                a_b = jnp.full((LANES,), a_row[m], jnp.float32)
                ql = q[L_IDX[m]]
                for fc in range(NFC):
                    new[j * NFC + fc] = new[j * NFC + fc] + a_b * ql[fc]
            return tuple(new)

        def run(lo, hi, cont):
            z = jnp.zeros((LANES,), jnp.float32)
            accs = tuple(jnp.where(cont, v, z) for v in load_accs())
            accs = lax.fori_loop(lo, hi, edge_body, accs)
            store_accs(accs)

        return run

    def compute(b, buf, r_cur, active):
        sidx_v, r_v, a_v, s_v, w_v, piece_v, sa, sb, sc = buf
        pass_fns = [make_pass(pi, a_v, s_v, w_v)
                    for pi in range(len(PASSES))]
        eb = b * EB
        lo_i = jnp.maximum(e_start - eb, 0)
        hi_i = jnp.minimum(e_end - eb, EB)
        npieces = piece_v[pl.ds(0, LANES)][0]
        npieces = jnp.where(active, npieces, 0)

        def piece_body(k, r_c):
            st = piece_v[pl.ds(1 + k, LANES)][0]
            en = piece_v[pl.ds(2 + k, LANES)][0]
            act = (st >= lo_i) & (st < hi_i)
            r_seg = r_v[pl.ds(st, LANES)][0]

            @pl.when(act)
            def _piece():
                @pl.when(r_seg != r_c)
                def _boundary():
                    pltpu.sync_copy(acc, out.at[r_c])
                    fill_rows(r_c + 1, r_seg)

                cont = r_seg == r_c
                for fn in pass_fns:
                    fn(st, en, cont)

            return jnp.where(act, r_seg, r_c)

        return lax.fori_loop(0, npieces, piece_body, r_cur)

    b_lo = e_start // EB
    b_hi = (e_end + EB - 1) // EB
    nblk = b_hi - b_lo

    @pl.when(nblk > 0)
    def _prime():
        prefetch_linear(b_lo, bufs[0])

    def pair_body(t, r_cur):
        b0 = b_lo + 2 * t
        b1 = b0 + 1
        wait_sidx_start_gather(b0, bufs[0])

        @pl.when(b1 < b_hi)
        def _pf1():
            prefetch_linear(b1, bufs[1])

        wait_rest(b0, bufs[0])
        r_cur = compute(b0, bufs[0], r_cur, True)

        act1 = b1 < b_hi

        @pl.when(act1)
        def _g1():
            wait_sidx_start_gather(b1, bufs[1])

        @pl.when(b0 + 2 < b_hi)
        def _pf0():
            prefetch_linear(b0 + 2, bufs[0])

        @pl.when(act1)
        def _w1():
            wait_rest(b1, bufs[1])

        return compute(b1, bufs[1], r_cur, act1)

    npairs = (nblk + 1) // 2
    r_cur = lax.fori_loop(0, npairs, pair_body, r_start)

    @pl.when(r_end > r_start)
    def _final():
        pltpu.sync_copy(acc, out.at[r_cur])
        fill_rows(r_cur + 1, r_end)


def kernel(node_feats, edge_attrs, tp_weights, sender_list, receiver_list,
           first_occurences):
    n, f = node_feats.shape
    e = edge_attrs.shape[0]

    fo_ext = jnp.concatenate(
        [first_occurences.astype(jnp.int32),
         jnp.array([e], jnp.int32)])
    targets = (jnp.arange(NW, dtype=jnp.int32) * (e // NW)).astype(jnp.int32)
    b = jnp.searchsorted(fo_ext, targets, side="left").astype(jnp.int32)
    b_ext = jnp.concatenate([b, jnp.array([n], jnp.int32)])
    e_starts = fo_ext[b_ext[:-1]]
    e_ends = fo_ext[b_ext[1:]]
    params = jnp.zeros((NW, 16), jnp.int32)
    params = (params.at[:, 0].set(e_starts)
                    .at[:, 1].set(e_ends)
                    .at[:, 2].set(b_ext[:-1])
                    .at[:, 3].set(b_ext[1:]))

    rl = receiver_list.astype(jnp.int32)
    nb = e // EB
    is_start = jnp.concatenate(
        [jnp.ones((1,), bool), rl[1:] != rl[:-1]])
    local = is_start | (jnp.arange(e) % EB == 0)
    l2 = local.reshape(nb, EB)
    slot = jnp.cumsum(l2.astype(jnp.int32), axis=1) - 1
    npieces = l2.sum(axis=1).astype(jnp.int32)
    starts = jnp.full((nb * (EB + 1) + 1,), EB, jnp.int32)
    rows = jnp.arange(e, dtype=jnp.int32) // EB
    flat_idx = rows * (EB + 1) + slot.reshape(-1)
    flat_idx = jnp.where(local.reshape(-1), flat_idx, nb * (EB + 1))
    starts = starts.at[flat_idx].set(jnp.arange(e, dtype=jnp.int32) % EB)
    starts = starts[:-1].reshape(nb, EB + 1)
    ptable = jnp.concatenate(
        [npieces[:, None], starts,
         jnp.full((nb, PTW - EB - 2), EB, jnp.int32)], axis=1).reshape(-1)

    mesh = plsc.VectorSubcoreMesh(core_axis_name="c", subcore_axis_name="s",
                                  num_cores=NC, num_subcores=NS)
    buf_types = [
        pltpu.VMEM((EB,), jnp.int32),
        pltpu.VMEM((EB + LANES,), jnp.int32),
        pltpu.VMEM((EB * 16,), jnp.float32),
        pltpu.VMEM((EB, f), jnp.float32),
        pltpu.VMEM((EB * 4 * f,), jnp.float32),
        pltpu.VMEM((PTW + LANES,), jnp.int32),
        pltpu.SemaphoreType.DMA,
        pltpu.SemaphoreType.DMA,
        pltpu.SemaphoreType.DMA,
    ]
    run = functools.partial(
        pl.kernel,
        out_type=jax.ShapeDtypeStruct((n, 16, f), jnp.float32),
        mesh=mesh,
        scratch_types=[
            pltpu.VMEM((LANES,), jnp.int32),
            pltpu.VMEM((16, f), jnp.float32),
            pltpu.VMEM((16, f), jnp.float32),
        ] + buf_types + buf_types,
    )(_sc_body)
    return run(node_feats, edge_attrs.reshape(-1), tp_weights.reshape(-1),
               sender_list.astype(jnp.int32), rl, ptable, params.reshape(-1))
